# Initial kernel scaffold; baseline (speedup 1.0000x reference)
import functools

import jax
import jax.numpy as jnp
from jax import lax
from jax.experimental import pallas as pl
from jax.experimental.pallas import tpu as pltpu
from jax.experimental.pallas import tpu_sc as plsc

VOCAB = 100000
CODE_DIM = 64
BATCH = 16384
SEQ = 200

B = BATCH * SEQ
SW = 128
NW = 32
KB = 10
ROWS_PER_CHUNK = KB * SW
ROWS_PER_WORKER = B // NW
CHUNKS_PER_WORKER = ROWS_PER_WORKER // ROWS_PER_CHUNK
STREAMS_PER_WORKER = ROWS_PER_WORKER // SW


def _gather_sc(idx2d, codebook):
    mesh = plsc.VectorSubcoreMesh(core_axis_name="c", subcore_axis_name="s")

    @functools.partial(
        pl.kernel,
        mesh=mesh,
        out_type=jax.ShapeDtypeStruct((B, CODE_DIM), jnp.float32),
        scratch_types=[
            pltpu.VMEM((KB, SW), jnp.int32),
            pltpu.VMEM((ROWS_PER_CHUNK, CODE_DIM), jnp.float32),
            pltpu.SemaphoreType.DMA,
        ],
    )
    def k(idx_hbm, table_hbm, out_hbm, idx_v, rows_v, sem):
        wid = lax.axis_index("s") * 2 + lax.axis_index("c")
        stream_base = wid * STREAMS_PER_WORKER

        def body(g, carry):
            srow = stream_base + g * KB
            pltpu.sync_copy(idx_hbm.at[pl.ds(srow, KB)], idx_v)
            copies = []
            for j in range(KB):
                copies.append(
                    pltpu.async_copy(
                        table_hbm.at[idx_v.at[j]],
                        rows_v.at[pl.ds(j * SW, SW)],
                        sem,
                    )
                )
            for c in copies:
                c.wait()
            pltpu.sync_copy(
                rows_v, out_hbm.at[pl.ds(srow * SW, ROWS_PER_CHUNK)]
            )
            return carry

        lax.fori_loop(0, CHUNKS_PER_WORKER, body, 0)

    return k(idx2d, codebook)


def kernel(x, codebook):
    idx2d = x.reshape(-1).astype(jnp.int32).reshape(B // SW, SW)
    out = _gather_sc(idx2d, codebook)
    return out.reshape(BATCH, SEQ, CODE_DIM)


# speedup vs baseline: 4.8692x; 4.8692x over previous
import functools

import numpy as np

import jax
import jax.numpy as jnp
from jax import lax
from jax.experimental import pallas as pl
from jax.experimental.pallas import tpu as pltpu
from jax.experimental.pallas import tpu_sc as plsc
from jax._src import config as _jcfg

VOCAB = 100000
CODE_DIM = 64
BATCH = 16384
SEQ = 200

B = BATCH * SEQ
SW = 128
NW = 32
KB = 8
ROWS_PER_CHUNK = KB * SW
ROWS_PER_WORKER = B // NW
CHUNKS_PER_WORKER = ROWS_PER_WORKER // ROWS_PER_CHUNK
STREAMS_PER_WORKER = ROWS_PER_WORKER // SW


def _gather_sc(idx2d, codebook):
    mesh = plsc.VectorSubcoreMesh(core_axis_name="c", subcore_axis_name="s")

    @functools.partial(
        pl.kernel,
        mesh=mesh,
        out_type=jax.ShapeDtypeStruct((B, CODE_DIM), jnp.float32),
        scratch_types=[
            pltpu.VMEM((KB, SW), jnp.int32),
            pltpu.VMEM((ROWS_PER_CHUNK, CODE_DIM), jnp.float32),
            pltpu.SemaphoreType.DMA,
        ],
        compiler_params=pltpu.CompilerParams(use_tc_tiling_on_sc=False),
    )
    def k(idx_hbm, table_hbm, out_hbm, idx_v, rows_v, sem):
        wid = lax.axis_index("s") * 2 + lax.axis_index("c")
        stream_base = wid * STREAMS_PER_WORKER

        @pl.loop(np.int32(0), np.int32(CHUNKS_PER_WORKER))
        def body(g):
            srow = stream_base + g * np.int32(KB)
            pltpu.sync_copy(idx_hbm.at[pl.ds(srow, KB)], idx_v)
            copies = []
            for j in range(KB):
                copies.append(
                    pltpu.async_copy(
                        table_hbm.at[idx_v.at[j]],
                        rows_v.at[pl.ds(j * SW, SW)],
                        sem,
                    )
                )
            for c in copies:
                c.wait()
            pltpu.sync_copy(
                rows_v, out_hbm.at[pl.ds(srow * np.int32(SW), ROWS_PER_CHUNK)]
            )

    return k(idx2d, codebook)


def kernel(x, codebook):
    idx2d = x.reshape(-1).astype(jnp.int32).reshape(B // SW, SW)
    with _jcfg.enable_x64(False):
        out = _gather_sc(idx2d, codebook)
    return out.reshape(BATCH, SEQ, CODE_DIM)


# speedup vs baseline: 5.0260x; 1.0322x over previous
import functools

import numpy as np

import jax
import jax.numpy as jnp
from jax import lax
from jax.experimental import pallas as pl
from jax.experimental.pallas import tpu as pltpu
from jax.experimental.pallas import tpu_sc as plsc
from jax._src import config as _jcfg

VOCAB = 100000
CODE_DIM = 64
BATCH = 16384
SEQ = 200

B = BATCH * SEQ
NW = 32
UR = 800
ROWS_PER_WORKER = B // NW
UNITS_PER_WORKER = ROWS_PER_WORKER // UR


def _gather_sc(idx, codebook):
    mesh = plsc.VectorSubcoreMesh(core_axis_name="c", subcore_axis_name="s")

    @functools.partial(
        pl.kernel,
        mesh=mesh,
        out_type=jax.ShapeDtypeStruct((B, CODE_DIM), jnp.float32),
        scratch_types=[
            pltpu.VMEM((UR,), jnp.int32),
            pltpu.VMEM((UR,), jnp.int32),
            pltpu.VMEM((UR, CODE_DIM), jnp.float32),
            pltpu.VMEM((UR, CODE_DIM), jnp.float32),
            pltpu.SemaphoreType.DMA,
            pltpu.SemaphoreType.DMA,
            pltpu.SemaphoreType.DMA,
            pltpu.SemaphoreType.DMA,
            pltpu.SemaphoreType.DMA,
            pltpu.SemaphoreType.DMA,
        ],
        compiler_params=pltpu.CompilerParams(use_tc_tiling_on_sc=False),
    )
    def k(idx_hbm, table_hbm, out_hbm, idx0, idx1, rows0, rows1,
          is0, is1, gs0, gs1, ws0, ws1):
        wid = lax.axis_index("s") * 2 + lax.axis_index("c")
        row_base = wid * ROWS_PER_WORKER
        idx_v = (idx0, idx1)
        rows_v = (rows0, rows1)
        isem = (is0, is1)
        gsem = (gs0, gs1)
        wsem = (ws0, ws1)

        def idx_copy(u, b):
            base = row_base + u * np.int32(UR)
            return pltpu.make_async_copy(
                idx_hbm.at[pl.ds(base, UR)], idx_v[b], isem[b]
            )

        def gather_copy(b):
            return pltpu.make_async_copy(
                table_hbm.at[idx_v[b]], rows_v[b], gsem[b]
            )

        def write_copy(u, b):
            base = row_base + u * np.int32(UR)
            return pltpu.make_async_copy(
                rows_v[b], out_hbm.at[pl.ds(base, UR)], wsem[b]
            )

        for b in range(2):
            idx_copy(np.int32(b), b).start()
        for b in range(2):
            idx_copy(np.int32(b), b).wait()
            gather_copy(b).start()
        for b in range(2):
            gather_copy(b).wait()
            idx_copy(np.int32(b + 2), b).start()
            write_copy(np.int32(b), b).start()

        @pl.loop(np.int32(2), np.int32(UNITS_PER_WORKER - 2), step=np.int32(2))
        def body(u0):
            for b in range(2):
                u = u0 + np.int32(b)
                write_copy(u - np.int32(2), b).wait()
                idx_copy(u, b).wait()
                gather_copy(b).start()
                gather_copy(b).wait()
                idx_copy(u + np.int32(2), b).start()
                write_copy(u, b).start()

        for b in range(2):
            u = np.int32(UNITS_PER_WORKER - 2 + b)
            write_copy(u - np.int32(2), b).wait()
            idx_copy(u, b).wait()
            gather_copy(b).start()
            gather_copy(b).wait()
            write_copy(u, b).start()
        for b in range(2):
            write_copy(np.int32(UNITS_PER_WORKER - 2 + b), b).wait()

    return k(idx, codebook)


def kernel(x, codebook):
    idx = x.reshape(-1).astype(jnp.int32)
    with _jcfg.enable_x64(False):
        out = _gather_sc(idx, codebook)
    return out.reshape(BATCH, SEQ, CODE_DIM)
